# Initial kernel scaffold; baseline (speedup 1.0000x reference)
import jax
import jax.numpy as jnp
from jax.experimental import pallas as pl


def kernel(x):
    raise NotImplementedError("write your pallas kernel here")


# speedup vs baseline: 1.7258x; 1.7258x over previous
import functools

import jax
import jax.numpy as jnp
from jax import lax
from jax.experimental import pallas as pl
from jax.experimental.pallas import tpu as pltpu
from jax.experimental.pallas import tpu_sc as plsc

_B = 128
_N = 2048
_L = 16
_NC = 2
_NS = 16
_NW = _NC * _NS
_RPW = _B // _NW
_CH = _N // _L

_NEG = float("-inf")


def _spa_body(x_hbm, out_hbm, xin, xout):
    wid = lax.axis_index("s") * _NC + lax.axis_index("c")
    base = wid * _RPW
    pltpu.sync_copy(x_hbm.at[pl.ds(base, _RPW)], xin)
    lane = lax.iota(jnp.int32, _L)

    for r in range(_RPW):
        def scan_body(c, carry):
            l1, l2, c1 = carry
            v = xin[r, pl.ds(c * _L, _L)]
            gt = v > l1
            c1 = jnp.where(gt, c, c1)
            l2 = jnp.maximum(l2, jnp.minimum(l1, v))
            l1 = jnp.maximum(l1, v)
            return l1, l2, c1

        l1, l2, c1 = lax.fori_loop(
            0, _CH, scan_body,
            (jnp.full((_L,), _NEG), jnp.full((_L,), _NEG),
             jnp.zeros((_L,), jnp.int32)))

        m1 = jnp.max(l1)
        lane_star = jnp.min(jnp.where(l1 == m1, lane, _L))
        sel = lane == lane_star
        cross = jnp.max(jnp.where(sel, _NEG, l1))
        l2_at = jnp.max(jnp.where(sel, l2, _NEG))
        c_at = jnp.max(jnp.where(sel, c1, -1))
        m2 = jnp.maximum(cross, l2_at)
        j_star = c_at * _L + lane_star
        p1v = jnp.full((_L,), jnp.maximum(m1, jnp.float32(0.0)))
        p2v = jnp.full((_L,), jnp.maximum(m2, jnp.float32(0.0)))

        def fill_body(c, _):
            xout[r, pl.ds(c * _L, _L)] = p1v
            return 0

        lax.fori_loop(0, _CH, fill_body, 0)
        plsc.store_scatter(
            xout,
            [jnp.full((_L,), r, jnp.int32), jnp.full((_L,), j_star, jnp.int32)],
            p2v, mask=lane == 0)

    pltpu.sync_copy(xout, out_hbm.at[pl.ds(base, _RPW)])


_spa_payment = functools.partial(
    pl.kernel,
    out_type=jax.ShapeDtypeStruct((_B, _N), jnp.float32),
    mesh=plsc.VectorSubcoreMesh(core_axis_name="c", subcore_axis_name="s"),
    scratch_types=[
        pltpu.VMEM((_RPW, _N), jnp.float32),
        pltpu.VMEM((_RPW, _N), jnp.float32),
    ],
    compiler_params=pltpu.CompilerParams(needs_layout_passes=False),
)(_spa_body)


def kernel(x):
    return _spa_payment(x)


# speedup vs baseline: 1.7565x; 1.0178x over previous
import functools

import jax
import jax.numpy as jnp
from jax import lax
from jax.experimental import pallas as pl
from jax.experimental.pallas import tpu as pltpu
from jax.experimental.pallas import tpu_sc as plsc

_B = 128
_N = 2048
_L = 16
_NC = 2
_NS = 16
_NW = _NC * _NS
_RPW = _B // _NW
_CH = _N // _L

_NEG = float("-inf")


def _spa_body(x_hbm, out_hbm, xin, xout, sem_in, sem_out):
    wid = lax.axis_index("s") * _NC + lax.axis_index("c")
    base = wid * _RPW
    lanes = lax.iota(jnp.int32, _L)

    in_cp = [
        pltpu.async_copy(
            x_hbm.at[pl.ds(base + r, 1)], xin.at[pl.ds(r, 1)], sem_in)
        for r in range(_RPW)
    ]
    out_cp = []
    for r in range(_RPW):
        in_cp[r].wait()

        l1 = jnp.full((_L,), _NEG)
        l2 = jnp.full((_L,), _NEG)
        for c in range(_CH):
            v = xin[r, pl.ds(c * _L, _L)]
            l2 = jnp.maximum(l2, jnp.minimum(l1, v))
            l1 = jnp.maximum(l1, v)

        s1, _ = plsc.sort_key_val(l1, lanes, descending=True)
        m1 = s1[0]
        m2 = jnp.maximum(s1[1], jnp.max(l2))
        m1v = jnp.full((_L,), m1)
        p1v = jnp.full((_L,), jnp.maximum(m1, jnp.float32(0.0)))
        p2v = jnp.full((_L,), jnp.maximum(m2, jnp.float32(0.0)))

        for c in range(_CH):
            v = xin[r, pl.ds(c * _L, _L)]
            xout[r, pl.ds(c * _L, _L)] = jnp.where(v == m1v, p2v, p1v)

        out_cp.append(
            pltpu.async_copy(
                xout.at[pl.ds(r, 1)], out_hbm.at[pl.ds(base + r, 1)],
                sem_out))
    for cp in out_cp:
        cp.wait()


_spa_payment = functools.partial(
    pl.kernel,
    out_type=jax.ShapeDtypeStruct((_B, _N), jnp.float32),
    mesh=plsc.VectorSubcoreMesh(core_axis_name="c", subcore_axis_name="s"),
    scratch_types=[
        pltpu.VMEM((_RPW, _N), jnp.float32),
        pltpu.VMEM((_RPW, _N), jnp.float32),
        pltpu.SemaphoreType.DMA,
        pltpu.SemaphoreType.DMA,
    ],
    compiler_params=pltpu.CompilerParams(needs_layout_passes=False),
)(_spa_body)


def kernel(x):
    return _spa_payment(x)


# speedup vs baseline: 1.9675x; 1.1202x over previous
import functools

import jax
import jax.numpy as jnp
from jax import lax
from jax.experimental import pallas as pl
from jax.experimental.pallas import tpu as pltpu
from jax.experimental.pallas import tpu_sc as plsc

_B = 128
_N = 2048
_L = 16
_NC = 2
_NS = 16
_NW = _NC * _NS
_RPW = _B // _NW
_CH = _N // _L
_U = 8

_NEG = float("-inf")


def _spa_body(x_hbm, out_hbm, xin, xout, sem_in, sem_out):
    wid = lax.axis_index("s") * _NC + lax.axis_index("c")
    base = wid * _RPW
    lanes = lax.iota(jnp.int32, _L)

    in_cp = [
        pltpu.async_copy(
            x_hbm.at[pl.ds(base + r, 1)], xin.at[pl.ds(r, 1)], sem_in)
        for r in range(_RPW)
    ]
    out_cp = []
    for r in range(_RPW):
        in_cp[r].wait()

        def scan_body(c, carry):
            l1, l2 = carry
            for k in range(_U):
                v = xin[r, pl.ds((c * _U + k) * _L, _L)]
                l2 = jnp.maximum(l2, jnp.minimum(l1, v))
                l1 = jnp.maximum(l1, v)
            return l1, l2

        l1, l2 = lax.fori_loop(
            0, _CH // _U, scan_body,
            (jnp.full((_L,), _NEG), jnp.full((_L,), _NEG)))

        s1, _ = plsc.sort_key_val(l1, lanes, descending=True)
        m1 = s1[0]
        m2 = jnp.maximum(s1[1], jnp.max(l2))
        m1v = jnp.full((_L,), m1)
        p1v = jnp.full((_L,), jnp.maximum(m1, jnp.float32(0.0)))
        p2v = jnp.full((_L,), jnp.maximum(m2, jnp.float32(0.0)))

        def fill_body(c, _):
            for k in range(_U):
                v = xin[r, pl.ds((c * _U + k) * _L, _L)]
                xout[r, pl.ds((c * _U + k) * _L, _L)] = jnp.where(
                    v == m1v, p2v, p1v)
            return 0

        lax.fori_loop(0, _CH // _U, fill_body, 0)

        out_cp.append(
            pltpu.async_copy(
                xout.at[pl.ds(r, 1)], out_hbm.at[pl.ds(base + r, 1)],
                sem_out))
    for cp in out_cp:
        cp.wait()


_spa_payment = functools.partial(
    pl.kernel,
    out_type=jax.ShapeDtypeStruct((_B, _N), jnp.float32),
    mesh=plsc.VectorSubcoreMesh(core_axis_name="c", subcore_axis_name="s"),
    scratch_types=[
        pltpu.VMEM((_RPW, _N), jnp.float32),
        pltpu.VMEM((_RPW, _N), jnp.float32),
        pltpu.SemaphoreType.DMA,
        pltpu.SemaphoreType.DMA,
    ],
    compiler_params=pltpu.CompilerParams(needs_layout_passes=False),
)(_spa_body)


def kernel(x):
    return _spa_payment(x)
